# baseline (speedup 1.0000x reference)
import functools

import jax
import jax.numpy as jnp
from jax import lax
from jax.experimental import pallas as pl
from jax.experimental.pallas import tpu as pltpu
from jax.experimental.pallas import tpu_sc as plsc

N = 16384
C = 64
H = 32
K = 16
NUM_B = 4
BR = 256
W = 512
NBLK = N // BR
INT_MAX = 2147483647
F32_INF = float("inf")


def _phase_a_body(clo_ref, chi_ref, coordT_ref, batchT_ref, coord_ref,
                  batch_ref, feat_ref, w1_ref, b1_ref,
                  topkd_ref, topki_ref, h_ref, bn_ref):
    i = pl.program_id(0)
    r0 = i * BR

    rc = coord_ref[...]
    br = batch_ref[...]
    sqr = jnp.sum(rc * rc, axis=1, keepdims=True)
    ri = r0 + lax.broadcasted_iota(jnp.int32, (BR, 1), 0)

    h = jnp.dot(feat_ref[...], w1_ref[...],
                preferred_element_type=jnp.float32) + b1_ref[...]
    h_ref[...] = h
    bt_row = batchT_ref[:, pl.ds(r0, BR)]
    onehotT = (lax.broadcasted_iota(jnp.int32, (NUM_B, 1), 0)
               == bt_row).astype(jnp.float32)

    @pl.when(i == 0)
    def _():
        bn_ref[...] = jnp.zeros_like(bn_ref)

    bn_ref[0:NUM_B, 0:H] += jnp.dot(onehotT, h,
                                    preferred_element_type=jnp.float32)
    bn_ref[NUM_B:2 * NUM_B, 0:H] += jnp.dot(
        onehotT, h * h, preferred_element_type=jnp.float32)
    bn_ref[2 * NUM_B:3 * NUM_B, 0:1] += jnp.sum(onehotT, axis=1,
                                                keepdims=True)

    clo = clo_ref[i]
    chi = chi_ref[i]
    t0 = clo // W
    t1 = (chi + W - 1) // W

    lane16 = lax.broadcasted_iota(jnp.int32, (1, K), 1)

    def tile_body(t, carry):
        cur_d, cur_i = carry
        cs = pl.multiple_of(t * W, W)
        ct = coordT_ref[:, pl.ds(cs, W)]
        bc = batchT_ref[:, pl.ds(cs, W)]
        sqc = jnp.sum(ct * ct, axis=0, keepdims=True)
        d2 = sqr + sqc - 2.0 * jnp.dot(rc, ct,
                                       preferred_element_type=jnp.float32)
        cj = cs + lax.broadcasted_iota(jnp.int32, (1, W), 1)
        valid = (bc == br) & (cj != ri)
        d2 = jnp.where(valid, jnp.maximum(d2, 0.0), F32_INF)

        kth = jnp.max(cur_d, axis=1, keepdims=True)
        cand = jnp.sum((d2 <= kth).astype(jnp.int32), axis=1, keepdims=True)
        steps = jnp.minimum(jnp.max(cand), K)

        def extract(s, ec):
            d2c, s_d, s_i = ec
            m = jnp.min(d2c, axis=1, keepdims=True)
            sel = jnp.min(jnp.where(d2c == m, cj, INT_MAX),
                          axis=1, keepdims=True)
            s_d = jnp.where(lane16 == s, m, s_d)
            s_i = jnp.where(lane16 == s, sel, s_i)
            d2c = jnp.where((d2c == m) & (cj == sel), F32_INF, d2c)
            return d2c, s_d, s_i

        _, tile_d, tile_i = lax.fori_loop(
            0, steps, extract,
            (d2, jnp.full((BR, K), F32_INF, jnp.float32),
             jnp.zeros((BR, K), jnp.int32)))

        new_d = jnp.zeros((BR, K), jnp.float32)
        new_i = jnp.zeros((BR, K), jnp.int32)
        for k in range(K):
            m1 = jnp.min(cur_d, axis=1, keepdims=True)
            m2 = jnp.min(tile_d, axis=1, keepdims=True)
            m = jnp.minimum(m1, m2)
            i1 = jnp.min(jnp.where(cur_d == m, cur_i, INT_MAX),
                         axis=1, keepdims=True)
            i2 = jnp.min(jnp.where(tile_d == m, tile_i, INT_MAX),
                         axis=1, keepdims=True)
            sel = jnp.minimum(i1, i2)
            new_d = jnp.where(lane16 == k, m, new_d)
            new_i = jnp.where(lane16 == k, sel, new_i)
            cur_d = jnp.where((cur_d == m) & (cur_i == sel), F32_INF, cur_d)
            tile_d = jnp.where((tile_d == m) & (tile_i == sel), F32_INF,
                               tile_d)
        return new_d, new_i

    init = (jnp.full((BR, K), F32_INF, jnp.float32),
            jnp.zeros((BR, K), jnp.int32))
    fin_d, fin_i = lax.fori_loop(t0, t1, tile_body, init)
    topkd_ref[...] = jnp.sqrt(fin_d)
    topki_ref[...] = fin_i


def _phase_a(clo, chi, coordT, batchT, coord2d, batch2d, feat, W1, b1_2):
    return pl.pallas_call(
        _phase_a_body,
        grid=(NBLK,),
        in_specs=[
            pl.BlockSpec(memory_space=pltpu.SMEM),
            pl.BlockSpec(memory_space=pltpu.SMEM),
            pl.BlockSpec((3, N), lambda i: (0, 0)),
            pl.BlockSpec((1, N), lambda i: (0, 0)),
            pl.BlockSpec((BR, 3), lambda i: (i, 0)),
            pl.BlockSpec((BR, 1), lambda i: (i, 0)),
            pl.BlockSpec((BR, C), lambda i: (i, 0)),
            pl.BlockSpec((C, H), lambda i: (0, 0)),
            pl.BlockSpec((1, H), lambda i: (0, 0)),
        ],
        out_specs=[
            pl.BlockSpec((BR, K), lambda i: (i, 0)),
            pl.BlockSpec((BR, K), lambda i: (i, 0)),
            pl.BlockSpec((BR, H), lambda i: (i, 0)),
            pl.BlockSpec((16, 128), lambda i: (0, 0)),
        ],
        out_shape=[
            jax.ShapeDtypeStruct((N, K), jnp.float32),
            jax.ShapeDtypeStruct((N, K), jnp.int32),
            jax.ShapeDtypeStruct((N, H), jnp.float32),
            jax.ShapeDtypeStruct((16, 128), jnp.float32),
        ],
    )(clo, chi, coordT, batchT, coord2d, batch2d, feat, W1, b1_2)


def _sc_gather(idx_flat, x, y, z):
    info = plsc.get_sparse_core_info()
    nc, ns = info.num_cores, info.num_subcores
    nw = nc * ns
    b_per_w = (N * K) // nw
    mesh = plsc.VectorSubcoreMesh(core_axis_name="c", subcore_axis_name="s")

    @functools.partial(
        pl.kernel, mesh=mesh,
        out_type=[jax.ShapeDtypeStruct((N * K,), jnp.float32)] * 3,
        scratch_types=[
            pltpu.VMEM((b_per_w,), jnp.int32),
            pltpu.VMEM((b_per_w,), jnp.float32),
            pltpu.VMEM((b_per_w,), jnp.float32),
            pltpu.VMEM((b_per_w,), jnp.float32),
            pltpu.SemaphoreType.DMA,
            pltpu.SemaphoreType.DMA,
            pltpu.SemaphoreType.DMA,
        ],
    )
    def gather_k(idx_hbm, x_hbm, y_hbm, z_hbm, gx_hbm, gy_hbm, gz_hbm,
                 idx_v, gxv, gyv, gzv, sem_x, sem_y, sem_z):
        wid = lax.axis_index("s") * nc + lax.axis_index("c")
        base = wid * b_per_w
        pltpu.sync_copy(idx_hbm.at[pl.ds(base, b_per_w)], idx_v)
        cx = pltpu.async_copy(x_hbm.at[idx_v], gxv, sem_x)
        cy = pltpu.async_copy(y_hbm.at[idx_v], gyv, sem_y)
        cz = pltpu.async_copy(z_hbm.at[idx_v], gzv, sem_z)
        cx.wait()
        cy.wait()
        cz.wait()
        pltpu.sync_copy(gxv, gx_hbm.at[pl.ds(base, b_per_w)])
        pltpu.sync_copy(gyv, gy_hbm.at[pl.ds(base, b_per_w)])
        pltpu.sync_copy(gzv, gz_hbm.at[pl.ds(base, b_per_w)])

    return gather_k(idx_flat, x, y, z)


def _acos(x):
    ax = jnp.abs(x)
    p = jnp.float32(-0.0012624911)
    p = p * ax + jnp.float32(0.0066700901)
    p = p * ax + jnp.float32(-0.0170881256)
    p = p * ax + jnp.float32(0.0308918810)
    p = p * ax + jnp.float32(-0.0501743046)
    p = p * ax + jnp.float32(0.0889789874)
    p = p * ax + jnp.float32(-0.2145988016)
    p = p * ax + jnp.float32(1.5707963050)
    a = jnp.sqrt(jnp.maximum(1.0 - ax, 0.0)) * p
    return jnp.where(x >= 0.0, a, jnp.float32(3.14159265358979) - a)


def _cos_small(t):
    t2 = t * t
    c = jnp.float32(1.0 / 40320.0)
    c = c * t2 - jnp.float32(1.0 / 720.0)
    c = c * t2 + jnp.float32(1.0 / 24.0)
    c = c * t2 - jnp.float32(0.5)
    c = c * t2 + jnp.float32(1.0)
    return c


def _phase_c_body(gx_ref, gy_ref, gz_ref, topkd_ref, h_ref, batch_ref,
                  bn_ref, gamma_ref, beta_ref, w2_ref, b2_ref, out_ref):
    gx = gx_ref[...]
    gy = gy_ref[...]
    gz = gz_ref[...]
    kf = jnp.float32(K)

    mx = jnp.sum(gx, axis=1, keepdims=True) / kf
    my = jnp.sum(gy, axis=1, keepdims=True) / kf
    mz = jnp.sum(gz, axis=1, keepdims=True) / kf
    den = jnp.float32(1.0 / (K - 1))
    cxx = (jnp.sum(gx * gx, axis=1, keepdims=True) - kf * mx * mx) * den
    cyy = (jnp.sum(gy * gy, axis=1, keepdims=True) - kf * my * my) * den
    czz = (jnp.sum(gz * gz, axis=1, keepdims=True) - kf * mz * mz) * den
    cxy = (jnp.sum(gx * gy, axis=1, keepdims=True) - kf * mx * my) * den
    cxz = (jnp.sum(gx * gz, axis=1, keepdims=True) - kf * mx * mz) * den
    cyz = (jnp.sum(gy * gz, axis=1, keepdims=True) - kf * my * mz) * den

    tr = cxx + cyy + czz
    q = tr * jnp.float32(1.0 / 3.0)
    p1 = cxy * cxy + cxz * cxz + cyz * cyz
    dxx, dyy, dzz = cxx - q, cyy - q, czz - q
    p2 = dxx * dxx + dyy * dyy + dzz * dzz + 2.0 * p1
    p = jnp.sqrt(jnp.maximum(p2 * jnp.float32(1.0 / 6.0), 0.0))
    ps = jnp.maximum(p, jnp.float32(1e-30))
    bxx, byy, bzz = dxx / ps, dyy / ps, dzz / ps
    bxy, bxz, byz = cxy / ps, cxz / ps, cyz / ps
    detb = (bxx * (byy * bzz - byz * byz)
            - bxy * (bxy * bzz - byz * bxz)
            + bxz * (bxy * byz - byy * bxz))
    r = jnp.clip(detb * 0.5, -1.0, 1.0)
    phi = _acos(r) * jnp.float32(1.0 / 3.0)
    lam_max = q + 2.0 * p * _cos_small(phi)
    linearity = (2.0 * lam_max - tr) / (tr + jnp.float32(1e-6))

    mean_dist = jnp.sum(topkd_ref[...], axis=1, keepdims=True) / kf
    density = 1.0 / (mean_dist + jnp.float32(1e-6))

    cnt = bn_ref[2 * NUM_B:3 * NUM_B, 0:1]
    cntf = jnp.maximum(cnt, 1.0)
    mu4 = bn_ref[0:NUM_B, 0:H] / cntf
    var4 = bn_ref[NUM_B:2 * NUM_B, 0:H] / cntf - mu4 * mu4

    brc = batch_ref[...]
    onehot = (brc == lax.broadcasted_iota(jnp.int32, (1, NUM_B), 1)
              ).astype(jnp.float32)
    mu_row = jnp.dot(onehot, mu4, preferred_element_type=jnp.float32)
    var_row = jnp.dot(onehot, var4, preferred_element_type=jnp.float32)
    cnt_row = jnp.dot(onehot, cnt, preferred_element_type=jnp.float32)

    hn = ((h_ref[...] - mu_row) / jnp.sqrt(var_row + jnp.float32(1e-5))
          * gamma_ref[...] + beta_ref[...])
    hn = jnp.maximum(hn, 0.0)
    logits = jnp.dot(hn, w2_ref[...],
                     preferred_element_type=jnp.float32) + b2_ref[...]
    lmax = jnp.max(logits, axis=1, keepdims=True)
    e = jnp.exp(logits - lmax)
    probs = e / jnp.sum(e, axis=1, keepdims=True)
    p0 = probs[:, 0:1]
    p1r = probs[:, 1:2]
    p2r = probs[:, 2:3]

    tower = (density * 2.0 + p0) * jnp.float32(1.0 / 3.0)
    bg = (jnp.maximum(1.0 - linearity, 1.0 - density) + p1r) * jnp.float32(1.0 / 3.0)
    line = (linearity * 2.0 + p2r) * jnp.float32(1.0 / 3.0)

    o0 = tower * 0.05 + bg * 0.2 + line * 0.1 + jnp.float32(1e-6)
    o2 = tower * 0.05 + bg * 0.2 + line * 2.0 + jnp.float32(1e-6)
    stacked = jnp.concatenate([o0, o0, o2], axis=1)
    ok = cnt_row >= jnp.float32(K)
    out_ref[...] = jnp.where(ok, stacked, jnp.float32(0.2))


def _phase_c(gx, gy, gz, topkd, h_all, batch2d, bn, gamma2, beta2, W2, b2_2):
    return pl.pallas_call(
        _phase_c_body,
        grid=(NBLK,),
        in_specs=[
            pl.BlockSpec((BR, K), lambda i: (i, 0)),
            pl.BlockSpec((BR, K), lambda i: (i, 0)),
            pl.BlockSpec((BR, K), lambda i: (i, 0)),
            pl.BlockSpec((BR, K), lambda i: (i, 0)),
            pl.BlockSpec((BR, H), lambda i: (i, 0)),
            pl.BlockSpec((BR, 1), lambda i: (i, 0)),
            pl.BlockSpec((16, 128), lambda i: (0, 0)),
            pl.BlockSpec((1, H), lambda i: (0, 0)),
            pl.BlockSpec((1, H), lambda i: (0, 0)),
            pl.BlockSpec((H, 3), lambda i: (0, 0)),
            pl.BlockSpec((1, 3), lambda i: (0, 0)),
        ],
        out_specs=pl.BlockSpec((BR, 3), lambda i: (i, 0)),
        out_shape=jax.ShapeDtypeStruct((N, 3), jnp.float32),
    )(gx, gy, gz, topkd, h_all, batch2d, bn, gamma2, beta2, W2, b2_2)


def kernel(feat, coord, batch, W1, b1, gamma, beta, W2, b2):
    batch32 = batch.astype(jnp.int32)
    offsets = jnp.searchsorted(
        batch32, jnp.arange(NUM_B + 1, dtype=jnp.int32)).astype(jnp.int32)
    r0s = jnp.arange(NBLK, dtype=jnp.int32) * BR
    b_lo = batch32[r0s]
    b_hi = batch32[r0s + (BR - 1)]
    clo = offsets[b_lo]
    chi = offsets[b_hi + 1]

    coordT = coord.T.astype(jnp.float32)
    batchT = batch32.reshape(1, N)
    batch2d = batch32.reshape(N, 1)

    topkd, topki, h_all, bn = _phase_a(
        clo, chi, coordT, batchT, coord, batch2d, feat, W1, b1.reshape(1, H))

    gx, gy, gz = _sc_gather(topki.reshape(N * K),
                            coord[:, 0], coord[:, 1], coord[:, 2])

    return _phase_c(gx.reshape(N, K), gy.reshape(N, K), gz.reshape(N, K),
                    topkd, h_all, batch2d, bn,
                    gamma.reshape(1, H), beta.reshape(1, H),
                    W2, b2.reshape(1, 3))


# speedup vs baseline: 2.9529x; 2.9529x over previous
import functools

import jax
import jax.numpy as jnp
from jax import lax
from jax.experimental import pallas as pl
from jax.experimental.pallas import tpu as pltpu
from jax.experimental.pallas import tpu_sc as plsc

N = 16384
C = 64
H = 32
K = 16
NUM_B = 4
BR = 256
W = 1024
NBLK = N // BR
INT_MAX = 2147483647
F32_INF = float("inf")


def _phase_a_body(clo_ref, chi_ref, coordT_ref, batchT_ref, coord_ref,
                  batch_ref, feat_ref, w1_ref, b1_ref,
                  topkd_ref, topki_ref, h_ref, bn_ref):
    i = pl.program_id(0)
    r0 = i * BR

    rc = coord_ref[...]
    br = batch_ref[...]
    sqr = jnp.sum(rc * rc, axis=1, keepdims=True)
    ri = r0 + lax.broadcasted_iota(jnp.int32, (BR, 1), 0)

    h = jnp.dot(feat_ref[...], w1_ref[...],
                preferred_element_type=jnp.float32) + b1_ref[...]
    h_ref[...] = h
    bt_row = batchT_ref[:, pl.ds(r0, BR)]
    onehotT = (lax.broadcasted_iota(jnp.int32, (NUM_B, 1), 0)
               == bt_row).astype(jnp.float32)

    @pl.when(i == 0)
    def _():
        bn_ref[...] = jnp.zeros_like(bn_ref)

    bn_ref[0:NUM_B, 0:H] += jnp.dot(onehotT, h,
                                    preferred_element_type=jnp.float32)
    bn_ref[NUM_B:2 * NUM_B, 0:H] += jnp.dot(
        onehotT, h * h, preferred_element_type=jnp.float32)
    bn_ref[2 * NUM_B:3 * NUM_B, 0:1] += jnp.sum(onehotT, axis=1,
                                                keepdims=True)

    clo = clo_ref[i]
    chi = chi_ref[i]
    t0 = clo // W
    t1 = (chi + W - 1) // W

    lane16 = lax.broadcasted_iota(jnp.int32, (1, K), 1)

    def tile_body(t, carry):
        cur_d, cur_i = carry
        cs = pl.multiple_of(t * W, W)
        ct = coordT_ref[:, pl.ds(cs, W)]
        bc = batchT_ref[:, pl.ds(cs, W)]
        sqc = jnp.sum(ct * ct, axis=0, keepdims=True)
        d2 = sqr + sqc - 2.0 * jnp.dot(rc, ct,
                                       preferred_element_type=jnp.float32)
        cj = cs + lax.broadcasted_iota(jnp.int32, (1, W), 1)
        valid = (bc == br) & (cj != ri)
        d2 = jnp.where(valid, jnp.maximum(d2, 0.0), F32_INF)

        new_d = jnp.zeros((BR, K), jnp.float32)
        new_i = jnp.zeros((BR, K), jnp.int32)
        for k in range(K):
            m1 = jnp.min(cur_d, axis=1, keepdims=True)
            m2 = jnp.min(d2, axis=1, keepdims=True)
            m = jnp.minimum(m1, m2)
            i1 = jnp.min(jnp.where(cur_d == m, cur_i, INT_MAX),
                         axis=1, keepdims=True)
            i2 = jnp.min(jnp.where(d2 == m, cj, INT_MAX),
                         axis=1, keepdims=True)
            sel = jnp.minimum(i1, i2)
            new_d = jnp.where(lane16 == k, m, new_d)
            new_i = jnp.where(lane16 == k, sel, new_i)
            cur_d = jnp.where((cur_d == m) & (cur_i == sel), F32_INF, cur_d)
            d2 = jnp.where((d2 == m) & (cj == sel), F32_INF, d2)
        return new_d, new_i

    init = (jnp.full((BR, K), F32_INF, jnp.float32),
            jnp.zeros((BR, K), jnp.int32))
    fin_d, fin_i = lax.fori_loop(t0, t1, tile_body, init)
    topkd_ref[...] = jnp.sqrt(fin_d)
    topki_ref[...] = fin_i


def _phase_a(clo, chi, coordT, batchT, coord2d, batch2d, feat, W1, b1_2):
    return pl.pallas_call(
        _phase_a_body,
        grid=(NBLK,),
        in_specs=[
            pl.BlockSpec(memory_space=pltpu.SMEM),
            pl.BlockSpec(memory_space=pltpu.SMEM),
            pl.BlockSpec((3, N), lambda i: (0, 0)),
            pl.BlockSpec((1, N), lambda i: (0, 0)),
            pl.BlockSpec((BR, 3), lambda i: (i, 0)),
            pl.BlockSpec((BR, 1), lambda i: (i, 0)),
            pl.BlockSpec((BR, C), lambda i: (i, 0)),
            pl.BlockSpec((C, H), lambda i: (0, 0)),
            pl.BlockSpec((1, H), lambda i: (0, 0)),
        ],
        out_specs=[
            pl.BlockSpec((BR, K), lambda i: (i, 0)),
            pl.BlockSpec((BR, K), lambda i: (i, 0)),
            pl.BlockSpec((BR, H), lambda i: (i, 0)),
            pl.BlockSpec((16, 128), lambda i: (0, 0)),
        ],
        out_shape=[
            jax.ShapeDtypeStruct((N, K), jnp.float32),
            jax.ShapeDtypeStruct((N, K), jnp.int32),
            jax.ShapeDtypeStruct((N, H), jnp.float32),
            jax.ShapeDtypeStruct((16, 128), jnp.float32),
        ],
    )(clo, chi, coordT, batchT, coord2d, batch2d, feat, W1, b1_2)


def _sc_gather(idx_flat, x, y, z):
    info = plsc.get_sparse_core_info()
    nc, ns = info.num_cores, info.num_subcores
    nw = nc * ns
    b_per_w = (N * K) // nw
    mesh = plsc.VectorSubcoreMesh(core_axis_name="c", subcore_axis_name="s")

    @functools.partial(
        pl.kernel, mesh=mesh,
        out_type=[jax.ShapeDtypeStruct((N * K,), jnp.float32)] * 3,
        scratch_types=[
            pltpu.VMEM((b_per_w,), jnp.int32),
            pltpu.VMEM((b_per_w,), jnp.float32),
            pltpu.VMEM((b_per_w,), jnp.float32),
            pltpu.VMEM((b_per_w,), jnp.float32),
            pltpu.SemaphoreType.DMA,
            pltpu.SemaphoreType.DMA,
            pltpu.SemaphoreType.DMA,
        ],
    )
    def gather_k(idx_hbm, x_hbm, y_hbm, z_hbm, gx_hbm, gy_hbm, gz_hbm,
                 idx_v, gxv, gyv, gzv, sem_x, sem_y, sem_z):
        wid = lax.axis_index("s") * nc + lax.axis_index("c")
        base = wid * b_per_w
        pltpu.sync_copy(idx_hbm.at[pl.ds(base, b_per_w)], idx_v)
        cx = pltpu.async_copy(x_hbm.at[idx_v], gxv, sem_x)
        cy = pltpu.async_copy(y_hbm.at[idx_v], gyv, sem_y)
        cz = pltpu.async_copy(z_hbm.at[idx_v], gzv, sem_z)
        cx.wait()
        cy.wait()
        cz.wait()
        pltpu.sync_copy(gxv, gx_hbm.at[pl.ds(base, b_per_w)])
        pltpu.sync_copy(gyv, gy_hbm.at[pl.ds(base, b_per_w)])
        pltpu.sync_copy(gzv, gz_hbm.at[pl.ds(base, b_per_w)])

    return gather_k(idx_flat, x, y, z)


def _acos(x):
    ax = jnp.abs(x)
    p = jnp.float32(-0.0012624911)
    p = p * ax + jnp.float32(0.0066700901)
    p = p * ax + jnp.float32(-0.0170881256)
    p = p * ax + jnp.float32(0.0308918810)
    p = p * ax + jnp.float32(-0.0501743046)
    p = p * ax + jnp.float32(0.0889789874)
    p = p * ax + jnp.float32(-0.2145988016)
    p = p * ax + jnp.float32(1.5707963050)
    a = jnp.sqrt(jnp.maximum(1.0 - ax, 0.0)) * p
    return jnp.where(x >= 0.0, a, jnp.float32(3.14159265358979) - a)


def _cos_small(t):
    t2 = t * t
    c = jnp.float32(1.0 / 40320.0)
    c = c * t2 - jnp.float32(1.0 / 720.0)
    c = c * t2 + jnp.float32(1.0 / 24.0)
    c = c * t2 - jnp.float32(0.5)
    c = c * t2 + jnp.float32(1.0)
    return c


def _phase_c_body(gx_ref, gy_ref, gz_ref, topkd_ref, h_ref, batch_ref,
                  bn_ref, gamma_ref, beta_ref, w2_ref, b2_ref, out_ref):
    gx = gx_ref[...]
    gy = gy_ref[...]
    gz = gz_ref[...]
    kf = jnp.float32(K)

    mx = jnp.sum(gx, axis=1, keepdims=True) / kf
    my = jnp.sum(gy, axis=1, keepdims=True) / kf
    mz = jnp.sum(gz, axis=1, keepdims=True) / kf
    den = jnp.float32(1.0 / (K - 1))
    cxx = (jnp.sum(gx * gx, axis=1, keepdims=True) - kf * mx * mx) * den
    cyy = (jnp.sum(gy * gy, axis=1, keepdims=True) - kf * my * my) * den
    czz = (jnp.sum(gz * gz, axis=1, keepdims=True) - kf * mz * mz) * den
    cxy = (jnp.sum(gx * gy, axis=1, keepdims=True) - kf * mx * my) * den
    cxz = (jnp.sum(gx * gz, axis=1, keepdims=True) - kf * mx * mz) * den
    cyz = (jnp.sum(gy * gz, axis=1, keepdims=True) - kf * my * mz) * den

    tr = cxx + cyy + czz
    q = tr * jnp.float32(1.0 / 3.0)
    p1 = cxy * cxy + cxz * cxz + cyz * cyz
    dxx, dyy, dzz = cxx - q, cyy - q, czz - q
    p2 = dxx * dxx + dyy * dyy + dzz * dzz + 2.0 * p1
    p = jnp.sqrt(jnp.maximum(p2 * jnp.float32(1.0 / 6.0), 0.0))
    ps = jnp.maximum(p, jnp.float32(1e-30))
    bxx, byy, bzz = dxx / ps, dyy / ps, dzz / ps
    bxy, bxz, byz = cxy / ps, cxz / ps, cyz / ps
    detb = (bxx * (byy * bzz - byz * byz)
            - bxy * (bxy * bzz - byz * bxz)
            + bxz * (bxy * byz - byy * bxz))
    r = jnp.clip(detb * 0.5, -1.0, 1.0)
    phi = _acos(r) * jnp.float32(1.0 / 3.0)
    lam_max = q + 2.0 * p * _cos_small(phi)
    linearity = (2.0 * lam_max - tr) / (tr + jnp.float32(1e-6))

    mean_dist = jnp.sum(topkd_ref[...], axis=1, keepdims=True) / kf
    density = 1.0 / (mean_dist + jnp.float32(1e-6))

    cnt = bn_ref[2 * NUM_B:3 * NUM_B, 0:1]
    cntf = jnp.maximum(cnt, 1.0)
    mu4 = bn_ref[0:NUM_B, 0:H] / cntf
    var4 = bn_ref[NUM_B:2 * NUM_B, 0:H] / cntf - mu4 * mu4

    brc = batch_ref[...]
    onehot = (brc == lax.broadcasted_iota(jnp.int32, (1, NUM_B), 1)
              ).astype(jnp.float32)
    mu_row = jnp.dot(onehot, mu4, preferred_element_type=jnp.float32)
    var_row = jnp.dot(onehot, var4, preferred_element_type=jnp.float32)
    cnt_row = jnp.dot(onehot, cnt, preferred_element_type=jnp.float32)

    hn = ((h_ref[...] - mu_row) / jnp.sqrt(var_row + jnp.float32(1e-5))
          * gamma_ref[...] + beta_ref[...])
    hn = jnp.maximum(hn, 0.0)
    logits = jnp.dot(hn, w2_ref[...],
                     preferred_element_type=jnp.float32) + b2_ref[...]
    lmax = jnp.max(logits, axis=1, keepdims=True)
    e = jnp.exp(logits - lmax)
    probs = e / jnp.sum(e, axis=1, keepdims=True)
    p0 = probs[:, 0:1]
    p1r = probs[:, 1:2]
    p2r = probs[:, 2:3]

    tower = (density * 2.0 + p0) * jnp.float32(1.0 / 3.0)
    bg = (jnp.maximum(1.0 - linearity, 1.0 - density) + p1r) * jnp.float32(1.0 / 3.0)
    line = (linearity * 2.0 + p2r) * jnp.float32(1.0 / 3.0)

    o0 = tower * 0.05 + bg * 0.2 + line * 0.1 + jnp.float32(1e-6)
    o2 = tower * 0.05 + bg * 0.2 + line * 2.0 + jnp.float32(1e-6)
    stacked = jnp.concatenate([o0, o0, o2], axis=1)
    ok = cnt_row >= jnp.float32(K)
    out_ref[...] = jnp.where(ok, stacked, jnp.float32(0.2))


def _phase_c(gx, gy, gz, topkd, h_all, batch2d, bn, gamma2, beta2, W2, b2_2):
    return pl.pallas_call(
        _phase_c_body,
        grid=(NBLK,),
        in_specs=[
            pl.BlockSpec((BR, K), lambda i: (i, 0)),
            pl.BlockSpec((BR, K), lambda i: (i, 0)),
            pl.BlockSpec((BR, K), lambda i: (i, 0)),
            pl.BlockSpec((BR, K), lambda i: (i, 0)),
            pl.BlockSpec((BR, H), lambda i: (i, 0)),
            pl.BlockSpec((BR, 1), lambda i: (i, 0)),
            pl.BlockSpec((16, 128), lambda i: (0, 0)),
            pl.BlockSpec((1, H), lambda i: (0, 0)),
            pl.BlockSpec((1, H), lambda i: (0, 0)),
            pl.BlockSpec((H, 3), lambda i: (0, 0)),
            pl.BlockSpec((1, 3), lambda i: (0, 0)),
        ],
        out_specs=pl.BlockSpec((BR, 3), lambda i: (i, 0)),
        out_shape=jax.ShapeDtypeStruct((N, 3), jnp.float32),
    )(gx, gy, gz, topkd, h_all, batch2d, bn, gamma2, beta2, W2, b2_2)


def kernel(feat, coord, batch, W1, b1, gamma, beta, W2, b2):
    batch32 = batch.astype(jnp.int32)
    offsets = jnp.searchsorted(
        batch32, jnp.arange(NUM_B + 1, dtype=jnp.int32)).astype(jnp.int32)
    r0s = jnp.arange(NBLK, dtype=jnp.int32) * BR
    b_lo = batch32[r0s]
    b_hi = batch32[r0s + (BR - 1)]
    clo = offsets[b_lo]
    chi = offsets[b_hi + 1]

    coordT = coord.T.astype(jnp.float32)
    batchT = batch32.reshape(1, N)
    batch2d = batch32.reshape(N, 1)

    topkd, topki, h_all, bn = _phase_a(
        clo, chi, coordT, batchT, coord, batch2d, feat, W1, b1.reshape(1, H))

    gx, gy, gz = _sc_gather(topki.reshape(N * K),
                            coord[:, 0], coord[:, 1], coord[:, 2])

    return _phase_c(gx.reshape(N, K), gy.reshape(N, K), gz.reshape(N, K),
                    topkd, h_all, batch2d, bn,
                    gamma.reshape(1, H), beta.reshape(1, H),
                    W2, b2.reshape(1, 3))


# speedup vs baseline: 3.0640x; 1.0376x over previous
import functools

import jax
import jax.numpy as jnp
from jax import lax
from jax.experimental import pallas as pl
from jax.experimental.pallas import tpu as pltpu
from jax.experimental.pallas import tpu_sc as plsc

N = 16384
C = 64
H = 32
K = 16
NUM_B = 4
BR = 256
W = 2048
NBLK = N // BR
INT_MAX = 2147483647
F32_INF = float("inf")


def _phase_a_body(clo_ref, chi_ref, coordT_ref, batchT_ref, coord_ref,
                  batch_ref, feat_ref, w1_ref, b1_ref,
                  topkd_ref, topki_ref, h_ref, bn_ref):
    i = pl.program_id(0)
    r0 = i * BR

    rc = coord_ref[...]
    br = batch_ref[...]
    sqr = jnp.sum(rc * rc, axis=1, keepdims=True)
    ri = r0 + lax.broadcasted_iota(jnp.int32, (BR, 1), 0)

    h = jnp.dot(feat_ref[...], w1_ref[...],
                preferred_element_type=jnp.float32) + b1_ref[...]
    h_ref[...] = h
    bt_row = batchT_ref[:, pl.ds(r0, BR)]
    onehotT = (lax.broadcasted_iota(jnp.int32, (NUM_B, 1), 0)
               == bt_row).astype(jnp.float32)

    @pl.when(i == 0)
    def _():
        bn_ref[...] = jnp.zeros_like(bn_ref)

    bn_ref[0:NUM_B, 0:H] += jnp.dot(onehotT, h,
                                    preferred_element_type=jnp.float32)
    bn_ref[NUM_B:2 * NUM_B, 0:H] += jnp.dot(
        onehotT, h * h, preferred_element_type=jnp.float32)
    bn_ref[2 * NUM_B:3 * NUM_B, 0:1] += jnp.sum(onehotT, axis=1,
                                                keepdims=True)

    clo = clo_ref[i]
    chi = chi_ref[i]
    t0 = clo // W
    t1 = (chi + W - 1) // W

    lane16 = lax.broadcasted_iota(jnp.int32, (1, K), 1)

    def tile_body(t, carry):
        cur_d, cur_i = carry
        cs = pl.multiple_of(t * W, W)
        ct = coordT_ref[:, pl.ds(cs, W)]
        bc = batchT_ref[:, pl.ds(cs, W)]
        sqc = jnp.sum(ct * ct, axis=0, keepdims=True)
        d2 = sqr + sqc - 2.0 * jnp.dot(rc, ct,
                                       preferred_element_type=jnp.float32)
        cj = cs + lax.broadcasted_iota(jnp.int32, (1, W), 1)
        valid = (bc == br) & (cj != ri)
        d2 = jnp.where(valid, jnp.maximum(d2, 0.0), F32_INF)

        new_d = jnp.zeros((BR, K), jnp.float32)
        new_i = jnp.zeros((BR, K), jnp.int32)
        for k in range(K):
            m1 = jnp.min(cur_d, axis=1, keepdims=True)
            m2 = jnp.min(d2, axis=1, keepdims=True)
            m = jnp.minimum(m1, m2)
            i1 = jnp.min(jnp.where(cur_d == m, cur_i, INT_MAX),
                         axis=1, keepdims=True)
            i2 = jnp.min(jnp.where(d2 == m, cj, INT_MAX),
                         axis=1, keepdims=True)
            sel = jnp.minimum(i1, i2)
            new_d = jnp.where(lane16 == k, m, new_d)
            new_i = jnp.where(lane16 == k, sel, new_i)
            cur_d = jnp.where((cur_d == m) & (cur_i == sel), F32_INF, cur_d)
            d2 = jnp.where((d2 == m) & (cj == sel), F32_INF, d2)
        return new_d, new_i

    init = (jnp.full((BR, K), F32_INF, jnp.float32),
            jnp.zeros((BR, K), jnp.int32))
    fin_d, fin_i = lax.fori_loop(t0, t1, tile_body, init)
    topkd_ref[...] = jnp.sqrt(fin_d)
    topki_ref[...] = fin_i


def _phase_a(clo, chi, coordT, batchT, coord2d, batch2d, feat, W1, b1_2):
    return pl.pallas_call(
        _phase_a_body,
        grid=(NBLK,),
        in_specs=[
            pl.BlockSpec(memory_space=pltpu.SMEM),
            pl.BlockSpec(memory_space=pltpu.SMEM),
            pl.BlockSpec((3, N), lambda i: (0, 0)),
            pl.BlockSpec((1, N), lambda i: (0, 0)),
            pl.BlockSpec((BR, 3), lambda i: (i, 0)),
            pl.BlockSpec((BR, 1), lambda i: (i, 0)),
            pl.BlockSpec((BR, C), lambda i: (i, 0)),
            pl.BlockSpec((C, H), lambda i: (0, 0)),
            pl.BlockSpec((1, H), lambda i: (0, 0)),
        ],
        out_specs=[
            pl.BlockSpec((BR, K), lambda i: (i, 0)),
            pl.BlockSpec((BR, K), lambda i: (i, 0)),
            pl.BlockSpec((BR, H), lambda i: (i, 0)),
            pl.BlockSpec((16, 128), lambda i: (0, 0)),
        ],
        out_shape=[
            jax.ShapeDtypeStruct((N, K), jnp.float32),
            jax.ShapeDtypeStruct((N, K), jnp.int32),
            jax.ShapeDtypeStruct((N, H), jnp.float32),
            jax.ShapeDtypeStruct((16, 128), jnp.float32),
        ],
    )(clo, chi, coordT, batchT, coord2d, batch2d, feat, W1, b1_2)


def _sc_gather(idx_flat, x, y, z):
    info = plsc.get_sparse_core_info()
    nc, ns = info.num_cores, info.num_subcores
    nw = nc * ns
    b_per_w = (N * K) // nw
    mesh = plsc.VectorSubcoreMesh(core_axis_name="c", subcore_axis_name="s")

    @functools.partial(
        pl.kernel, mesh=mesh,
        out_type=[jax.ShapeDtypeStruct((N * K,), jnp.float32)] * 3,
        scratch_types=[
            pltpu.VMEM((b_per_w,), jnp.int32),
            pltpu.VMEM((b_per_w,), jnp.float32),
            pltpu.VMEM((b_per_w,), jnp.float32),
            pltpu.VMEM((b_per_w,), jnp.float32),
            pltpu.SemaphoreType.DMA,
            pltpu.SemaphoreType.DMA,
            pltpu.SemaphoreType.DMA,
        ],
    )
    def gather_k(idx_hbm, x_hbm, y_hbm, z_hbm, gx_hbm, gy_hbm, gz_hbm,
                 idx_v, gxv, gyv, gzv, sem_x, sem_y, sem_z):
        wid = lax.axis_index("s") * nc + lax.axis_index("c")
        base = wid * b_per_w
        pltpu.sync_copy(idx_hbm.at[pl.ds(base, b_per_w)], idx_v)
        cx = pltpu.async_copy(x_hbm.at[idx_v], gxv, sem_x)
        cy = pltpu.async_copy(y_hbm.at[idx_v], gyv, sem_y)
        cz = pltpu.async_copy(z_hbm.at[idx_v], gzv, sem_z)
        cx.wait()
        cy.wait()
        cz.wait()
        pltpu.sync_copy(gxv, gx_hbm.at[pl.ds(base, b_per_w)])
        pltpu.sync_copy(gyv, gy_hbm.at[pl.ds(base, b_per_w)])
        pltpu.sync_copy(gzv, gz_hbm.at[pl.ds(base, b_per_w)])

    return gather_k(idx_flat, x, y, z)


def _acos(x):
    ax = jnp.abs(x)
    p = jnp.float32(-0.0012624911)
    p = p * ax + jnp.float32(0.0066700901)
    p = p * ax + jnp.float32(-0.0170881256)
    p = p * ax + jnp.float32(0.0308918810)
    p = p * ax + jnp.float32(-0.0501743046)
    p = p * ax + jnp.float32(0.0889789874)
    p = p * ax + jnp.float32(-0.2145988016)
    p = p * ax + jnp.float32(1.5707963050)
    a = jnp.sqrt(jnp.maximum(1.0 - ax, 0.0)) * p
    return jnp.where(x >= 0.0, a, jnp.float32(3.14159265358979) - a)


def _cos_small(t):
    t2 = t * t
    c = jnp.float32(1.0 / 40320.0)
    c = c * t2 - jnp.float32(1.0 / 720.0)
    c = c * t2 + jnp.float32(1.0 / 24.0)
    c = c * t2 - jnp.float32(0.5)
    c = c * t2 + jnp.float32(1.0)
    return c


def _phase_c_body(gx_ref, gy_ref, gz_ref, topkd_ref, h_ref, batch_ref,
                  bn_ref, gamma_ref, beta_ref, w2_ref, b2_ref, out_ref):
    gx = gx_ref[...]
    gy = gy_ref[...]
    gz = gz_ref[...]
    kf = jnp.float32(K)

    mx = jnp.sum(gx, axis=1, keepdims=True) / kf
    my = jnp.sum(gy, axis=1, keepdims=True) / kf
    mz = jnp.sum(gz, axis=1, keepdims=True) / kf
    den = jnp.float32(1.0 / (K - 1))
    cxx = (jnp.sum(gx * gx, axis=1, keepdims=True) - kf * mx * mx) * den
    cyy = (jnp.sum(gy * gy, axis=1, keepdims=True) - kf * my * my) * den
    czz = (jnp.sum(gz * gz, axis=1, keepdims=True) - kf * mz * mz) * den
    cxy = (jnp.sum(gx * gy, axis=1, keepdims=True) - kf * mx * my) * den
    cxz = (jnp.sum(gx * gz, axis=1, keepdims=True) - kf * mx * mz) * den
    cyz = (jnp.sum(gy * gz, axis=1, keepdims=True) - kf * my * mz) * den

    tr = cxx + cyy + czz
    q = tr * jnp.float32(1.0 / 3.0)
    p1 = cxy * cxy + cxz * cxz + cyz * cyz
    dxx, dyy, dzz = cxx - q, cyy - q, czz - q
    p2 = dxx * dxx + dyy * dyy + dzz * dzz + 2.0 * p1
    p = jnp.sqrt(jnp.maximum(p2 * jnp.float32(1.0 / 6.0), 0.0))
    ps = jnp.maximum(p, jnp.float32(1e-30))
    bxx, byy, bzz = dxx / ps, dyy / ps, dzz / ps
    bxy, bxz, byz = cxy / ps, cxz / ps, cyz / ps
    detb = (bxx * (byy * bzz - byz * byz)
            - bxy * (bxy * bzz - byz * bxz)
            + bxz * (bxy * byz - byy * bxz))
    r = jnp.clip(detb * 0.5, -1.0, 1.0)
    phi = _acos(r) * jnp.float32(1.0 / 3.0)
    lam_max = q + 2.0 * p * _cos_small(phi)
    linearity = (2.0 * lam_max - tr) / (tr + jnp.float32(1e-6))

    mean_dist = jnp.sum(topkd_ref[...], axis=1, keepdims=True) / kf
    density = 1.0 / (mean_dist + jnp.float32(1e-6))

    cnt = bn_ref[2 * NUM_B:3 * NUM_B, 0:1]
    cntf = jnp.maximum(cnt, 1.0)
    mu4 = bn_ref[0:NUM_B, 0:H] / cntf
    var4 = bn_ref[NUM_B:2 * NUM_B, 0:H] / cntf - mu4 * mu4

    brc = batch_ref[...]
    onehot = (brc == lax.broadcasted_iota(jnp.int32, (1, NUM_B), 1)
              ).astype(jnp.float32)
    mu_row = jnp.dot(onehot, mu4, preferred_element_type=jnp.float32)
    var_row = jnp.dot(onehot, var4, preferred_element_type=jnp.float32)
    cnt_row = jnp.dot(onehot, cnt, preferred_element_type=jnp.float32)

    hn = ((h_ref[...] - mu_row) / jnp.sqrt(var_row + jnp.float32(1e-5))
          * gamma_ref[...] + beta_ref[...])
    hn = jnp.maximum(hn, 0.0)
    logits = jnp.dot(hn, w2_ref[...],
                     preferred_element_type=jnp.float32) + b2_ref[...]
    lmax = jnp.max(logits, axis=1, keepdims=True)
    e = jnp.exp(logits - lmax)
    probs = e / jnp.sum(e, axis=1, keepdims=True)
    p0 = probs[:, 0:1]
    p1r = probs[:, 1:2]
    p2r = probs[:, 2:3]

    tower = (density * 2.0 + p0) * jnp.float32(1.0 / 3.0)
    bg = (jnp.maximum(1.0 - linearity, 1.0 - density) + p1r) * jnp.float32(1.0 / 3.0)
    line = (linearity * 2.0 + p2r) * jnp.float32(1.0 / 3.0)

    o0 = tower * 0.05 + bg * 0.2 + line * 0.1 + jnp.float32(1e-6)
    o2 = tower * 0.05 + bg * 0.2 + line * 2.0 + jnp.float32(1e-6)
    stacked = jnp.concatenate([o0, o0, o2], axis=1)
    ok = cnt_row >= jnp.float32(K)
    out_ref[...] = jnp.where(ok, stacked, jnp.float32(0.2))


def _phase_c(gx, gy, gz, topkd, h_all, batch2d, bn, gamma2, beta2, W2, b2_2):
    return pl.pallas_call(
        _phase_c_body,
        grid=(NBLK,),
        in_specs=[
            pl.BlockSpec((BR, K), lambda i: (i, 0)),
            pl.BlockSpec((BR, K), lambda i: (i, 0)),
            pl.BlockSpec((BR, K), lambda i: (i, 0)),
            pl.BlockSpec((BR, K), lambda i: (i, 0)),
            pl.BlockSpec((BR, H), lambda i: (i, 0)),
            pl.BlockSpec((BR, 1), lambda i: (i, 0)),
            pl.BlockSpec((16, 128), lambda i: (0, 0)),
            pl.BlockSpec((1, H), lambda i: (0, 0)),
            pl.BlockSpec((1, H), lambda i: (0, 0)),
            pl.BlockSpec((H, 3), lambda i: (0, 0)),
            pl.BlockSpec((1, 3), lambda i: (0, 0)),
        ],
        out_specs=pl.BlockSpec((BR, 3), lambda i: (i, 0)),
        out_shape=jax.ShapeDtypeStruct((N, 3), jnp.float32),
    )(gx, gy, gz, topkd, h_all, batch2d, bn, gamma2, beta2, W2, b2_2)


def kernel(feat, coord, batch, W1, b1, gamma, beta, W2, b2):
    batch32 = batch.astype(jnp.int32)
    offsets = jnp.searchsorted(
        batch32, jnp.arange(NUM_B + 1, dtype=jnp.int32)).astype(jnp.int32)
    r0s = jnp.arange(NBLK, dtype=jnp.int32) * BR
    b_lo = batch32[r0s]
    b_hi = batch32[r0s + (BR - 1)]
    clo = offsets[b_lo]
    chi = offsets[b_hi + 1]

    coordT = coord.T.astype(jnp.float32)
    batchT = batch32.reshape(1, N)
    batch2d = batch32.reshape(N, 1)

    topkd, topki, h_all, bn = _phase_a(
        clo, chi, coordT, batchT, coord, batch2d, feat, W1, b1.reshape(1, H))

    gx, gy, gz = _sc_gather(topki.reshape(N * K),
                            coord[:, 0], coord[:, 1], coord[:, 2])

    return _phase_c(gx.reshape(N, K), gy.reshape(N, K), gz.reshape(N, K),
                    topkd, h_all, batch2d, bn,
                    gamma.reshape(1, H), beta.reshape(1, H),
                    W2, b2.reshape(1, 3))
